# Initial kernel scaffold; baseline (speedup 1.0000x reference)
import jax
import jax.numpy as jnp
from jax.experimental import pallas as pl


def kernel(x, edge_index, edge_attr, W1, b1, We1, be1, W2, b2, We2, be2, W3, b3, We3, be3, Wu, bu, g1, bt1, g2, bt2, g3, bt3, ge, bte, Wn, bnb, Wle, ble):
    raise NotImplementedError("write your pallas kernel here")


# speedup vs baseline: 2.0958x; 2.0958x over previous
import functools

import jax
import jax.numpy as jnp
from jax import lax
from jax.experimental import pallas as pl
from jax.experimental.pallas import tpu as pltpu
from jax.experimental.pallas import tpu_sc as plsc

N = 10000
E = 320000
D = 128
DE = 16

NC = 2
NS = 16
NW = NC * NS
CH = 128
CPW = -(-E // (NW * CH))
EPAD = NW * CH * CPW
NPAD = 10240
ZST = 1000
EPS = 1e-5

_MESH = plsc.VectorSubcoreMesh(core_axis_name="c", subcore_axis_name="s")
_SC_PARAMS = pltpu.CompilerParams(needs_layout_passes=False)


@functools.partial(
    pl.kernel,
    out_type=jax.ShapeDtypeStruct((NW * NPAD,), jnp.float32),
    mesh=_MESH,
    compiler_params=_SC_PARAMS,
    scratch_types=[
        pltpu.VMEM((NPAD,), jnp.float32),
        pltpu.VMEM((CH,), jnp.int32),
        pltpu.VMEM((CH,), jnp.float32),
    ],
)
def _deg_kernel(row_hbm, w_hbm, out_hbm, deg_v, idx_v, w_v):
    cid = lax.axis_index("c")
    sid = lax.axis_index("s")
    wid = sid * NC + cid

    def zero_body(i, _):
        deg_v[pl.ds(i * 16, 16)] = jnp.zeros((16,), jnp.float32)
        return 0
    lax.fori_loop(0, NPAD // 16, zero_body, 0)

    def chunk(c, _):
        base = (wid * CPW + c) * CH
        pltpu.sync_copy(row_hbm.at[pl.ds(base, CH)], idx_v)
        pltpu.sync_copy(w_hbm.at[pl.ds(base, CH)], w_v)

        def grp(g, _):
            ids = idx_v[pl.ds(g * 16, 16)]
            vals = w_v[pl.ds(g * 16, 16)]
            plsc.addupdate_scatter(deg_v, [ids], vals)
            return 0
        lax.fori_loop(0, CH // 16, grp, 0)
        return 0
    lax.fori_loop(0, CPW, chunk, 0)
    pltpu.sync_copy(deg_v, out_hbm.at[pl.ds(wid * NPAD, NPAD)])


@functools.partial(
    pl.kernel,
    out_type=jax.ShapeDtypeStruct((EPAD,), jnp.float32),
    mesh=_MESH,
    compiler_params=_SC_PARAMS,
    scratch_types=[
        pltpu.VMEM((NPAD,), jnp.float32),
        pltpu.VMEM((CH,), jnp.int32),
        pltpu.VMEM((CH,), jnp.int32),
        pltpu.VMEM((CH,), jnp.float32),
        pltpu.VMEM((CH,), jnp.float32),
    ],
)
def _norm_kernel(dinv_hbm, row_hbm, col_hbm, w_hbm, out_hbm,
                 dinv_v, r_v, c_v, w_v, o_v):
    cid = lax.axis_index("c")
    sid = lax.axis_index("s")
    wid = sid * NC + cid
    pltpu.sync_copy(dinv_hbm, dinv_v)

    def chunk(ck, _):
        base = (wid * CPW + ck) * CH
        pltpu.sync_copy(row_hbm.at[pl.ds(base, CH)], r_v)
        pltpu.sync_copy(col_hbm.at[pl.ds(base, CH)], c_v)
        pltpu.sync_copy(w_hbm.at[pl.ds(base, CH)], w_v)

        def grp(g, _):
            a = plsc.load_gather(dinv_v, [r_v[pl.ds(g * 16, 16)]])
            b = plsc.load_gather(dinv_v, [c_v[pl.ds(g * 16, 16)]])
            o_v[pl.ds(g * 16, 16)] = a * b * w_v[pl.ds(g * 16, 16)]
            return 0
        lax.fori_loop(0, CH // 16, grp, 0)
        pltpu.sync_copy(o_v, out_hbm.at[pl.ds(base, CH)])
        return 0
    lax.fori_loop(0, CPW, chunk, 0)


@functools.partial(
    pl.kernel,
    out_type=jax.ShapeDtypeStruct((NC, N, D), jnp.float32),
    mesh=_MESH,
    compiler_params=_SC_PARAMS,
    scratch_types=[
        pltpu.VMEM_SHARED((N, D), jnp.float32),
        pltpu.VMEM((CH,), jnp.int32),
        pltpu.VMEM((CH,), jnp.int32),
        pltpu.VMEM((CH,), jnp.float32),
        pltpu.VMEM((CH, D), jnp.float32),
        pltpu.VMEM((CH, D), jnp.float32),
        pltpu.SemaphoreType.DMA,
    ],
)
def _layer_kernel(h_hbm, e_hbm, row_hbm, col_hbm, norm_hbm, zero_hbm, out_hbm,
                  acc_sh, r_v, c_v, n_v, e_v, g_v, sem):
    cid = lax.axis_index("c")
    sid = lax.axis_index("s")
    wid = sid * NC + cid
    @pl.when(sid < 10)
    def _():
        pltpu.sync_copy(zero_hbm.at[pl.ds(sid * ZST, ZST)],
                        acc_sh.at[pl.ds(sid * ZST, ZST)])
    plsc.subcore_barrier()

    def chunk(ck, _):
        base = (wid * CPW + ck) * CH
        pltpu.sync_copy(row_hbm.at[pl.ds(base, CH)], r_v)
        pltpu.sync_copy(col_hbm.at[pl.ds(base, CH)], c_v)
        pltpu.sync_copy(norm_hbm.at[pl.ds(base, CH)], n_v)
        pltpu.sync_copy(e_hbm.at[pl.ds(base, CH)], e_v)
        pltpu.async_copy(h_hbm.at[r_v], g_v, sem).wait()

        def edge(i, _):
            nv = plsc.load_gather(n_v, [jnp.full((16,), i, jnp.int32)])
            for v in range(D // 16):
                hv = g_v[i, pl.ds(v * 16, 16)]
                ev = e_v[i, pl.ds(v * 16, 16)]
                e_v[i, pl.ds(v * 16, 16)] = jnp.maximum(hv + ev, 0.0) * nv
            return 0
        lax.fori_loop(0, CH, edge, 0)
        pltpu.sync_copy(e_v, acc_sh.at[c_v], add=True)
        return 0
    lax.fori_loop(0, CPW, chunk, 0)
    plsc.subcore_barrier()

    @pl.when(sid == 0)
    def _():
        pltpu.sync_copy(acc_sh, out_hbm.at[cid])


@functools.partial(
    pl.kernel,
    out_type=jax.ShapeDtypeStruct((EPAD, D), jnp.float32),
    mesh=_MESH,
    compiler_params=_SC_PARAMS,
    scratch_types=[
        pltpu.VMEM((CH,), jnp.int32),
        pltpu.VMEM((CH,), jnp.int32),
        pltpu.VMEM((CH, D), jnp.float32),
        pltpu.VMEM((CH, D), jnp.float32),
        pltpu.SemaphoreType.DMA,
        pltpu.SemaphoreType.DMA,
    ],
)
def _pair_kernel(nx_hbm, row_hbm, col_hbm, out_hbm, r_v, c_v, a_v, b_v, s1, s2):
    cid = lax.axis_index("c")
    sid = lax.axis_index("s")
    wid = sid * NC + cid

    def chunk(ck, _):
        base = (wid * CPW + ck) * CH
        pltpu.sync_copy(row_hbm.at[pl.ds(base, CH)], r_v)
        pltpu.sync_copy(col_hbm.at[pl.ds(base, CH)], c_v)
        ca = pltpu.async_copy(nx_hbm.at[r_v], a_v, s1)
        cb = pltpu.async_copy(nx_hbm.at[c_v], b_v, s2)
        ca.wait()
        cb.wait()

        def edge(i, _):
            for v in range(D // 16):
                a_v[i, pl.ds(v * 16, 16)] = (a_v[i, pl.ds(v * 16, 16)]
                                             + b_v[i, pl.ds(v * 16, 16)])
            return 0
        lax.fori_loop(0, CH, edge, 0)
        pltpu.sync_copy(a_v, out_hbm.at[pl.ds(base, CH)])
        return 0
    lax.fori_loop(0, CPW, chunk, 0)


def _dinv_tc(deg_parts):
    def body(d_ref, o_ref):
        o_ref[...] = lax.rsqrt(jnp.sum(d_ref[...], axis=0) + 1.0)
    return pl.pallas_call(
        body,
        out_shape=jax.ShapeDtypeStruct((NPAD,), jnp.float32),
    )(deg_parts)


def _elin_tc(ea, w1t, b1, w2t, b2, w3t, b3):
    B = 512
    def body(ea_ref, w1_ref, b1_ref, w2_ref, b2_ref, w3_ref, b3_ref,
             o1_ref, o2_ref, o3_ref):
        ea_b = ea_ref[...]
        o1_ref[...] = jnp.dot(ea_b, w1_ref[...],
                              preferred_element_type=jnp.float32) + b1_ref[...]
        o2_ref[...] = jnp.dot(ea_b, w2_ref[...],
                              preferred_element_type=jnp.float32) + b2_ref[...]
        o3_ref[...] = jnp.dot(ea_b, w3_ref[...],
                              preferred_element_type=jnp.float32) + b3_ref[...]
    full = lambda s: pl.BlockSpec(s, lambda i: (0, 0))
    return pl.pallas_call(
        body,
        grid=(EPAD // B,),
        in_specs=[pl.BlockSpec((B, DE), lambda i: (i, 0)),
                  full((DE, D)), full((1, D)),
                  full((DE, D)), full((1, D)),
                  full((DE, D)), full((1, D))],
        out_specs=[pl.BlockSpec((B, D), lambda i: (i, 0))] * 3,
        out_shape=[jax.ShapeDtypeStruct((EPAD, D), jnp.float32)] * 3,
    )(ea, w1t, b1, w2t, b2, w3t, b3)


def _nlin_tc(x, wt, b):
    B = 1000
    def body(x_ref, w_ref, b_ref, o_ref):
        o_ref[...] = jnp.dot(x_ref[...], w_ref[...],
                             preferred_element_type=jnp.float32) + b_ref[...]
    return pl.pallas_call(
        body,
        grid=(N // B,),
        in_specs=[pl.BlockSpec((B, x.shape[1]), lambda i: (i, 0)),
                  pl.BlockSpec((x.shape[1], D), lambda i: (0, 0)),
                  pl.BlockSpec((1, D), lambda i: (0, 0))],
        out_specs=pl.BlockSpec((B, D), lambda i: (i, 0)),
        out_shape=jax.ShapeDtypeStruct((N, D), jnp.float32),
    )(x, wt, b)


def _nstats_tc(p):
    B = 1000
    def body(p_ref, st_ref, nx_ref):
        i = pl.program_id(0)
        z = p_ref[0] + p_ref[1]
        nx_ref[...] = z
        r = jnp.maximum(z, 0.0)
        st = jnp.concatenate([jnp.sum(r, axis=0, keepdims=True),
                              jnp.sum(r * r, axis=0, keepdims=True)], axis=0)
        @pl.when(i == 0)
        def _():
            st_ref[...] = st
        @pl.when(i > 0)
        def _():
            st_ref[...] = st_ref[...] + st
    return pl.pallas_call(
        body,
        grid=(N // B,),
        in_specs=[pl.BlockSpec((NC, B, D), lambda i: (0, i, 0))],
        out_specs=[pl.BlockSpec((2, D), lambda i: (0, 0)),
                   pl.BlockSpec((B, D), lambda i: (i, 0))],
        out_shape=[jax.ShapeDtypeStruct((2, D), jnp.float32),
                   jax.ShapeDtypeStruct((N, D), jnp.float32)],
    )(p)


def _comb2_tc(p, st, g, bt, wt, b):
    B = 1000
    def body(p_ref, st_ref, g_ref, bt_ref, w_ref, b_ref, o_ref):
        r = jnp.maximum(p_ref[0] + p_ref[1], 0.0)
        m = st_ref[0:1] * (1.0 / N)
        var = st_ref[1:2] * (1.0 / N) - m * m
        a = g_ref[...] * lax.rsqrt(var + EPS)
        c = bt_ref[...] - m * a
        o_ref[...] = jnp.dot(r * a + c, w_ref[...],
                             preferred_element_type=jnp.float32) + b_ref[...]
    full = lambda s: pl.BlockSpec(s, lambda i: (0, 0))
    return pl.pallas_call(
        body,
        grid=(N // B,),
        in_specs=[pl.BlockSpec((NC, B, D), lambda i: (0, i, 0)),
                  full((2, D)), full((1, D)), full((1, D)),
                  full((D, D)), full((1, D))],
        out_specs=pl.BlockSpec((B, D), lambda i: (i, 0)),
        out_shape=jax.ShapeDtypeStruct((N, D), jnp.float32),
    )(p, st, g, bt, wt, b)


def _newe_tc(s_mat, e3, wut, bu):
    B = 512
    def body(s_ref, e_ref, w_ref, b_ref, o_ref, st_ref):
        i = pl.program_id(0)
        z = jnp.dot(s_ref[...], w_ref[...],
                    preferred_element_type=jnp.float32) + b_ref[...] + e_ref[...]
        o_ref[...] = z
        r = jnp.maximum(z, 0.0)
        st = jnp.concatenate([jnp.sum(r, axis=0, keepdims=True),
                              jnp.sum(r * r, axis=0, keepdims=True)], axis=0)
        @pl.when(i == 0)
        def _():
            st_ref[...] = st
        @pl.when(i > 0)
        def _():
            st_ref[...] = st_ref[...] + st
    full = lambda s: pl.BlockSpec(s, lambda i: (0, 0))
    return pl.pallas_call(
        body,
        grid=(E // B,),
        in_specs=[pl.BlockSpec((B, D), lambda i: (i, 0)),
                  pl.BlockSpec((B, D), lambda i: (i, 0)),
                  full((D, D)), full((1, D))],
        out_specs=[pl.BlockSpec((B, D), lambda i: (i, 0)),
                   pl.BlockSpec((2, D), lambda i: (0, 0))],
        out_shape=[jax.ShapeDtypeStruct((E, D), jnp.float32),
                   jax.ShapeDtypeStruct((2, D), jnp.float32)],
    )(s_mat, e3, wut, bu)


def _comb1_tc(x, st, g, bt, wt, b, cnt):
    B = 512
    M = x.shape[0]
    def body(x_ref, st_ref, g_ref, bt_ref, w_ref, b_ref, o_ref):
        r = jnp.maximum(x_ref[...], 0.0)
        m = st_ref[0:1] * (1.0 / cnt)
        var = st_ref[1:2] * (1.0 / cnt) - m * m
        a = g_ref[...] * lax.rsqrt(var + EPS)
        c = bt_ref[...] - m * a
        o_ref[...] = jnp.dot(r * a + c, w_ref[...],
                             preferred_element_type=jnp.float32) + b_ref[...]
    full = lambda s: pl.BlockSpec(s, lambda i: (0, 0))
    return pl.pallas_call(
        body,
        grid=(M // B,),
        in_specs=[pl.BlockSpec((B, D), lambda i: (i, 0)),
                  full((2, D)), full((1, D)), full((1, D)),
                  full((D, D)), full((1, D))],
        out_specs=pl.BlockSpec((B, D), lambda i: (i, 0)),
        out_shape=jax.ShapeDtypeStruct((M, D), jnp.float32),
    )(x, st, g, bt, wt, b)


def kernel(x, edge_index, edge_attr, W1, b1, We1, be1, W2, b2, We2, be2,
           W3, b3, We3, be3, Wu, bu, g1, bt1, g2, bt2, g3, bt3, ge, bte,
           Wn, bnb, Wle, ble):
    pad = EPAD - E
    rowp = jnp.concatenate([edge_index[0], jnp.zeros((pad,), jnp.int32)])
    colp = jnp.concatenate([edge_index[1], jnp.zeros((pad,), jnp.int32)])
    w = jnp.concatenate([jnp.ones((E,), jnp.float32),
                         jnp.zeros((pad,), jnp.float32)])
    eap = jnp.concatenate([edge_attr, jnp.zeros((pad, DE), jnp.float32)], axis=0)
    zero_nd = jnp.zeros((N, D), jnp.float32)
    r2 = lambda v: v.reshape(1, D)

    deg_parts = _deg_kernel(rowp, w)
    dinv = _dinv_tc(deg_parts.reshape(NW, NPAD))
    norm = _norm_kernel(dinv, rowp, colp, w)
    e1, e2, e3 = _elin_tc(eap, We1.T, r2(be1), We2.T, r2(be2), We3.T, r2(be3))

    h1 = _nlin_tc(x, W1.T, r2(b1))
    p1 = _layer_kernel(h1, e1, rowp, colp, norm, zero_nd)
    st1, _ = _nstats_tc(p1)
    h2 = _comb2_tc(p1, st1, r2(g1), r2(bt1), W2.T, r2(b2))
    p2 = _layer_kernel(h2, e2, rowp, colp, norm, zero_nd)
    st2, _ = _nstats_tc(p2)
    h3 = _comb2_tc(p2, st2, r2(g2), r2(bt2), W3.T, r2(b3))
    p3 = _layer_kernel(h3, e3, rowp, colp, norm, zero_nd)
    st3, nx3 = _nstats_tc(p3)
    node = _comb2_tc(p3, st3, r2(g3), r2(bt3), Wn.T, r2(bnb))

    s_mat = _pair_kernel(nx3, rowp, colp)
    new_e, est = _newe_tc(s_mat, e3, Wu.T, r2(bu))
    edge = _comb1_tc(new_e, est, r2(ge), r2(bte), Wle.T, r2(ble), E)
    return (node, edge)


# speedup vs baseline: 2.7110x; 1.2935x over previous
import functools

import jax
import jax.numpy as jnp
from jax import lax
from jax.experimental import pallas as pl
from jax.experimental.pallas import tpu as pltpu
from jax.experimental.pallas import tpu_sc as plsc

N = 10000
E = 320000
D = 128
DE = 16

NC = 2
NS = 16
NW = NC * NS
CH = 48
CPW = 212
EW = CPW * CH
EPAD = NW * EW
NPAD = 10240
ZST = 1000
EPS = 1e-5

_MESH = plsc.VectorSubcoreMesh(core_axis_name="c", subcore_axis_name="s")
_SC_PARAMS = pltpu.CompilerParams(needs_layout_passes=False)


@functools.partial(
    pl.kernel,
    out_type=jax.ShapeDtypeStruct((NW * NPAD,), jnp.float32),
    mesh=_MESH,
    compiler_params=_SC_PARAMS,
    scratch_types=[
        pltpu.VMEM((NPAD,), jnp.float32),
        pltpu.VMEM((EW,), jnp.int32),
        pltpu.VMEM((EW,), jnp.float32),
    ],
)
def _deg_kernel(row_hbm, w_hbm, out_hbm, deg_v, idx_v, w_v):
    cid = lax.axis_index("c")
    sid = lax.axis_index("s")
    wid = sid * NC + cid

    def zero_body(i, _):
        deg_v[pl.ds(i * 16, 16)] = jnp.zeros((16,), jnp.float32)
        return 0
    lax.fori_loop(0, NPAD // 16, zero_body, 0)

    pltpu.sync_copy(row_hbm.at[wid], idx_v)
    pltpu.sync_copy(w_hbm.at[wid], w_v)

    def grp(g, _):
        ids = idx_v[pl.ds(g * 16, 16)]
        vals = w_v[pl.ds(g * 16, 16)]
        plsc.addupdate_scatter(deg_v, [ids], vals)
        return 0
    lax.fori_loop(0, EW // 16, grp, 0)
    pltpu.sync_copy(deg_v, out_hbm.at[pl.ds(wid * NPAD, NPAD)])


@functools.partial(
    pl.kernel,
    out_type=jax.ShapeDtypeStruct((NW, EW), jnp.float32),
    mesh=_MESH,
    compiler_params=_SC_PARAMS,
    scratch_types=[
        pltpu.VMEM((NPAD,), jnp.float32),
        pltpu.VMEM((EW,), jnp.int32),
        pltpu.VMEM((EW,), jnp.int32),
        pltpu.VMEM((EW,), jnp.float32),
        pltpu.VMEM((EW,), jnp.float32),
    ],
)
def _norm_kernel(dinv_hbm, row_hbm, col_hbm, w_hbm, out_hbm,
                 dinv_v, r_v, c_v, w_v, o_v):
    cid = lax.axis_index("c")
    sid = lax.axis_index("s")
    wid = sid * NC + cid
    pltpu.sync_copy(dinv_hbm, dinv_v)
    pltpu.sync_copy(row_hbm.at[wid], r_v)
    pltpu.sync_copy(col_hbm.at[wid], c_v)
    pltpu.sync_copy(w_hbm.at[wid], w_v)

    def grp(g, _):
        a = plsc.load_gather(dinv_v, [r_v[pl.ds(g * 16, 16)]])
        b = plsc.load_gather(dinv_v, [c_v[pl.ds(g * 16, 16)]])
        o_v[pl.ds(g * 16, 16)] = a * b * w_v[pl.ds(g * 16, 16)]
        return 0
    lax.fori_loop(0, EW // 16, grp, 0)
    pltpu.sync_copy(o_v, out_hbm.at[wid])


@functools.partial(
    pl.kernel,
    out_type=jax.ShapeDtypeStruct((NC, N, D), jnp.float32),
    mesh=_MESH,
    compiler_params=_SC_PARAMS,
    scratch_types=[
        pltpu.VMEM_SHARED((N, D), jnp.float32),
        pltpu.VMEM((3, CH), jnp.int32),
        pltpu.VMEM((3, CH), jnp.int32),
        pltpu.VMEM((3, CH), jnp.int32),
        pltpu.VMEM((3, CH), jnp.int32),
        pltpu.VMEM((CH, D), jnp.float32),
        pltpu.VMEM((CH, D), jnp.float32),
        pltpu.VMEM((CH, D), jnp.float32),
        pltpu.VMEM((CH, D), jnp.float32),
        pltpu.VMEM((CH, D), jnp.float32),
        pltpu.VMEM((CH, D), jnp.float32),
        pltpu.SemaphoreType.DMA,
        pltpu.SemaphoreType.DMA,
        pltpu.SemaphoreType.DMA,
        pltpu.SemaphoreType.DMA,
        pltpu.SemaphoreType.DMA,
        pltpu.SemaphoreType.DMA,
        pltpu.SemaphoreType.DMA,
        pltpu.SemaphoreType.DMA,
        pltpu.SemaphoreType.DMA,
        pltpu.SemaphoreType.DMA,
    ],
)
def _layer_kernel(h_hbm, e_hbm, pk_hbm, zero_hbm, out_hbm,
                  acc_sh, pk0, pk1, pk2, pk3, e_a, e_b, g_a, g_b, m_a, m_b,
                  sp0, sp1, sp2, sp3, sea, seb, sga, sgb, ssa, ssb):
    cid = lax.axis_index("c")
    sid = lax.axis_index("s")
    wid = sid * NC + cid
    wbase = wid * EW
    @pl.when(sid < 10)
    def _():
        pltpu.sync_copy(zero_hbm.at[pl.ds(sid * ZST, ZST)],
                        acc_sh.at[pl.ds(sid * ZST, ZST)])
    plsc.subcore_barrier()

    pks = (pk0, pk1, pk2, pk3)
    sps = (sp0, sp1, sp2, sp3)
    slots = ((e_a, g_a, m_a, sea, sga, ssa),
             (e_b, g_b, m_b, seb, sgb, ssb))

    def pk_fetch(ck, qi):
        pltpu.async_copy(pk_hbm.at[wid, ck], pks[qi], sps[qi])

    def pk_wait(ck, qi):
        pltpu.make_async_copy(pk_hbm.at[wid, ck], pks[qi], sps[qi]).wait()

    def eg_fetch(ck, qi, j):
        ebuf, gbuf, _, se, sg, _ = slots[j]
        pltpu.async_copy(e_hbm.at[pl.ds(wbase + ck * CH, CH)], ebuf, se)
        pltpu.async_copy(h_hbm.at[pks[qi].at[0]], gbuf, sg)

    pk_fetch(0, 0)
    pk_fetch(1, 1)
    pk_wait(0, 0)
    eg_fetch(0, 0, 0)

    def quad(k, _):
        for q in range(4):
            ck = 4 * k + q
            j = q % 2
            ebuf, gbuf, mbuf, se, sg, ss = slots[j]
            @pl.when(ck >= 2)
            def _():
                pltpu.make_async_copy(
                    mbuf, acc_sh.at[pks[(q + 2) % 4].at[1]], ss).wait()
            @pl.when(ck + 1 < CPW)
            def _():
                pk_wait(ck + 1, (q + 1) % 4)
                eg_fetch(ck + 1, (q + 1) % 4, (j + 1) % 2)
            @pl.when(ck + 2 < CPW)
            def _():
                pk_fetch(ck + 2, (q + 2) % 4)
            pltpu.make_async_copy(
                e_hbm.at[pl.ds(wbase + ck * CH, CH)], ebuf, se).wait()
            pltpu.make_async_copy(h_hbm.at[pks[q].at[0]], gbuf, sg).wait()

            def edge(i, _):
                nvi = plsc.load_gather(
                    pks[q], [jnp.full((16,), 2, jnp.int32),
                             jnp.full((16,), i, jnp.int32)])
                nv = plsc.bitcast(nvi, jnp.float32)
                for v in range(D // 16):
                    hv = gbuf[i, pl.ds(v * 16, 16)]
                    ev = ebuf[i, pl.ds(v * 16, 16)]
                    mbuf[i, pl.ds(v * 16, 16)] = (
                        jnp.maximum(hv + ev, 0.0) * nv)
                return 0
            lax.fori_loop(0, CH, edge, 0)
            pltpu.make_async_copy(
                mbuf, acc_sh.at[pks[q].at[1]], ss).start(add=True)
        return 0
    lax.fori_loop(0, CPW // 4, quad, 0)
    pltpu.make_async_copy(m_a, acc_sh.at[pks[2].at[1]], ssa).wait()
    pltpu.make_async_copy(m_b, acc_sh.at[pks[3].at[1]], ssb).wait()
    plsc.subcore_barrier()

    @pl.when(sid == 0)
    def _():
        pltpu.sync_copy(acc_sh, out_hbm.at[cid])


@functools.partial(
    pl.kernel,
    out_type=jax.ShapeDtypeStruct((EPAD, D), jnp.float32),
    mesh=_MESH,
    compiler_params=_SC_PARAMS,
    scratch_types=[
        pltpu.VMEM((CPW, CH), jnp.int32),
        pltpu.VMEM((CPW, CH), jnp.int32),
        pltpu.VMEM((CH, D), jnp.float32),
        pltpu.VMEM((CH, D), jnp.float32),
        pltpu.VMEM((CH, D), jnp.float32),
        pltpu.VMEM((CH, D), jnp.float32),
        pltpu.VMEM((CH, D), jnp.float32),
        pltpu.VMEM((CH, D), jnp.float32),
        pltpu.SemaphoreType.DMA,
        pltpu.SemaphoreType.DMA,
        pltpu.SemaphoreType.DMA,
        pltpu.SemaphoreType.DMA,
        pltpu.SemaphoreType.DMA,
        pltpu.SemaphoreType.DMA,
    ],
)
def _pair_kernel(nx_hbm, row_hbm, col_hbm, out_hbm,
                 ri_v, ci_v, a_a, a_b, b_a, b_b, o_a, o_b,
                 sra, srb, sca, scb, soa, sob):
    cid = lax.axis_index("c")
    sid = lax.axis_index("s")
    wid = sid * NC + cid
    wbase = wid * EW
    pltpu.sync_copy(row_hbm.at[wid], ri_v)
    pltpu.sync_copy(col_hbm.at[wid], ci_v)

    slots = ((a_a, b_a, o_a, sra, sca, soa),
             (a_b, b_b, o_b, srb, scb, sob))

    def fetch(ck, abuf, bbuf, sr, sc):
        pltpu.async_copy(nx_hbm.at[ri_v.at[ck]], abuf, sr)
        pltpu.async_copy(nx_hbm.at[ci_v.at[ck]], bbuf, sc)

    fetch(0, a_a, b_a, sra, sca)
    fetch(1, a_b, b_b, srb, scb)

    def pair(k, _):
        for j in (0, 1):
            abuf, bbuf, obuf, sr, sc, so = slots[j]
            ck = 2 * k + j
            pltpu.make_async_copy(nx_hbm.at[ri_v.at[ck]], abuf, sr).wait()
            pltpu.make_async_copy(nx_hbm.at[ci_v.at[ck]], bbuf, sc).wait()
            @pl.when(ck >= 2)
            def _():
                pltpu.make_async_copy(
                    obuf, out_hbm.at[pl.ds(wbase + (ck - 2) * CH, CH)],
                    so).wait()

            def edge(i, _):
                for v in range(D // 16):
                    obuf[i, pl.ds(v * 16, 16)] = (
                        abuf[i, pl.ds(v * 16, 16)]
                        + bbuf[i, pl.ds(v * 16, 16)])
                return 0
            lax.fori_loop(0, CH, edge, 0)
            pltpu.async_copy(
                obuf, out_hbm.at[pl.ds(wbase + ck * CH, CH)], so)
            @pl.when(ck + 2 < CPW)
            def _():
                fetch(ck + 2, abuf, bbuf, sr, sc)
        return 0
    lax.fori_loop(0, CPW // 2, pair, 0)
    pltpu.make_async_copy(
        o_a, out_hbm.at[pl.ds(wbase + (CPW - 2) * CH, CH)], soa).wait()
    pltpu.make_async_copy(
        o_b, out_hbm.at[pl.ds(wbase + (CPW - 1) * CH, CH)], sob).wait()


def _dinv_tc(deg_parts):
    def body(d_ref, o_ref):
        o_ref[...] = lax.rsqrt(jnp.sum(d_ref[...], axis=0) + 1.0)
    return pl.pallas_call(
        body,
        out_shape=jax.ShapeDtypeStruct((NPAD,), jnp.float32),
    )(deg_parts)


def _elin_tc(ea, w1t, b1, w2t, b2, w3t, b3):
    B = 512
    def body(ea_ref, w1_ref, b1_ref, w2_ref, b2_ref, w3_ref, b3_ref,
             o1_ref, o2_ref, o3_ref):
        ea_b = ea_ref[...]
        o1_ref[...] = jnp.dot(ea_b, w1_ref[...],
                              preferred_element_type=jnp.float32) + b1_ref[...]
        o2_ref[...] = jnp.dot(ea_b, w2_ref[...],
                              preferred_element_type=jnp.float32) + b2_ref[...]
        o3_ref[...] = jnp.dot(ea_b, w3_ref[...],
                              preferred_element_type=jnp.float32) + b3_ref[...]
    full = lambda s: pl.BlockSpec(s, lambda i: (0, 0))
    return pl.pallas_call(
        body,
        grid=(EPAD // B,),
        in_specs=[pl.BlockSpec((B, DE), lambda i: (i, 0)),
                  full((DE, D)), full((1, D)),
                  full((DE, D)), full((1, D)),
                  full((DE, D)), full((1, D))],
        out_specs=[pl.BlockSpec((B, D), lambda i: (i, 0))] * 3,
        out_shape=[jax.ShapeDtypeStruct((EPAD, D), jnp.float32)] * 3,
    )(ea, w1t, b1, w2t, b2, w3t, b3)


def _nlin_tc(x, wt, b):
    B = 1000
    def body(x_ref, w_ref, b_ref, o_ref):
        o_ref[...] = jnp.dot(x_ref[...], w_ref[...],
                             preferred_element_type=jnp.float32) + b_ref[...]
    return pl.pallas_call(
        body,
        grid=(N // B,),
        in_specs=[pl.BlockSpec((B, x.shape[1]), lambda i: (i, 0)),
                  pl.BlockSpec((x.shape[1], D), lambda i: (0, 0)),
                  pl.BlockSpec((1, D), lambda i: (0, 0))],
        out_specs=pl.BlockSpec((B, D), lambda i: (i, 0)),
        out_shape=jax.ShapeDtypeStruct((N, D), jnp.float32),
    )(x, wt, b)


def _nstats_tc(p):
    B = 1000
    def body(p_ref, st_ref, nx_ref):
        i = pl.program_id(0)
        z = p_ref[0] + p_ref[1]
        nx_ref[...] = z
        r = jnp.maximum(z, 0.0)
        st = jnp.concatenate([jnp.sum(r, axis=0, keepdims=True),
                              jnp.sum(r * r, axis=0, keepdims=True)], axis=0)
        @pl.when(i == 0)
        def _():
            st_ref[...] = st
        @pl.when(i > 0)
        def _():
            st_ref[...] = st_ref[...] + st
    return pl.pallas_call(
        body,
        grid=(N // B,),
        in_specs=[pl.BlockSpec((NC, B, D), lambda i: (0, i, 0))],
        out_specs=[pl.BlockSpec((2, D), lambda i: (0, 0)),
                   pl.BlockSpec((B, D), lambda i: (i, 0))],
        out_shape=[jax.ShapeDtypeStruct((2, D), jnp.float32),
                   jax.ShapeDtypeStruct((N, D), jnp.float32)],
    )(p)


def _comb2_tc(p, st, g, bt, wt, b):
    B = 1000
    def body(p_ref, st_ref, g_ref, bt_ref, w_ref, b_ref, o_ref):
        r = jnp.maximum(p_ref[0] + p_ref[1], 0.0)
        m = st_ref[0:1] * (1.0 / N)
        var = st_ref[1:2] * (1.0 / N) - m * m
        a = g_ref[...] * lax.rsqrt(var + EPS)
        c = bt_ref[...] - m * a
        o_ref[...] = jnp.dot(r * a + c, w_ref[...],
                             preferred_element_type=jnp.float32) + b_ref[...]
    full = lambda s: pl.BlockSpec(s, lambda i: (0, 0))
    return pl.pallas_call(
        body,
        grid=(N // B,),
        in_specs=[pl.BlockSpec((NC, B, D), lambda i: (0, i, 0)),
                  full((2, D)), full((1, D)), full((1, D)),
                  full((D, D)), full((1, D))],
        out_specs=pl.BlockSpec((B, D), lambda i: (i, 0)),
        out_shape=jax.ShapeDtypeStruct((N, D), jnp.float32),
    )(p, st, g, bt, wt, b)


def _newe_tc(s_mat, e3, wut, bu):
    B = 512
    def body(s_ref, e_ref, w_ref, b_ref, o_ref, st_ref):
        i = pl.program_id(0)
        z = jnp.dot(s_ref[...], w_ref[...],
                    preferred_element_type=jnp.float32) + b_ref[...] + e_ref[...]
        o_ref[...] = z
        r = jnp.maximum(z, 0.0)
        st = jnp.concatenate([jnp.sum(r, axis=0, keepdims=True),
                              jnp.sum(r * r, axis=0, keepdims=True)], axis=0)
        @pl.when(i == 0)
        def _():
            st_ref[...] = st
        @pl.when(i > 0)
        def _():
            st_ref[...] = st_ref[...] + st
    full = lambda s: pl.BlockSpec(s, lambda i: (0, 0))
    return pl.pallas_call(
        body,
        grid=(E // B,),
        in_specs=[pl.BlockSpec((B, D), lambda i: (i, 0)),
                  pl.BlockSpec((B, D), lambda i: (i, 0)),
                  full((D, D)), full((1, D))],
        out_specs=[pl.BlockSpec((B, D), lambda i: (i, 0)),
                   pl.BlockSpec((2, D), lambda i: (0, 0))],
        out_shape=[jax.ShapeDtypeStruct((E, D), jnp.float32),
                   jax.ShapeDtypeStruct((2, D), jnp.float32)],
    )(s_mat, e3, wut, bu)


def _comb1_tc(x, st, g, bt, wt, b, cnt):
    B = 512
    M = x.shape[0]
    def body(x_ref, st_ref, g_ref, bt_ref, w_ref, b_ref, o_ref):
        r = jnp.maximum(x_ref[...], 0.0)
        m = st_ref[0:1] * (1.0 / cnt)
        var = st_ref[1:2] * (1.0 / cnt) - m * m
        a = g_ref[...] * lax.rsqrt(var + EPS)
        c = bt_ref[...] - m * a
        o_ref[...] = jnp.dot(r * a + c, w_ref[...],
                             preferred_element_type=jnp.float32) + b_ref[...]
    full = lambda s: pl.BlockSpec(s, lambda i: (0, 0))
    return pl.pallas_call(
        body,
        grid=(M // B,),
        in_specs=[pl.BlockSpec((B, D), lambda i: (i, 0)),
                  full((2, D)), full((1, D)), full((1, D)),
                  full((D, D)), full((1, D))],
        out_specs=pl.BlockSpec((B, D), lambda i: (i, 0)),
        out_shape=jax.ShapeDtypeStruct((M, D), jnp.float32),
    )(x, st, g, bt, wt, b)


def kernel(x, edge_index, edge_attr, W1, b1, We1, be1, W2, b2, We2, be2,
           W3, b3, We3, be3, Wu, bu, g1, bt1, g2, bt2, g3, bt3, ge, bte,
           Wn, bnb, Wle, ble):
    pad = EPAD - E
    rowp = jnp.concatenate([edge_index[0], jnp.zeros((pad,), jnp.int32)])
    colp = jnp.concatenate([edge_index[1], jnp.zeros((pad,), jnp.int32)])
    w = jnp.concatenate([jnp.ones((E,), jnp.float32),
                         jnp.zeros((pad,), jnp.float32)])
    eap = jnp.concatenate([edge_attr, jnp.zeros((pad, DE), jnp.float32)], axis=0)
    zero_nd = jnp.zeros((N, D), jnp.float32)
    r2 = lambda v: v.reshape(1, D)
    row2 = rowp.reshape(NW, EW)
    col2 = colp.reshape(NW, EW)
    w2 = w.reshape(NW, EW)
    row3 = rowp.reshape(NW, CPW, CH)
    col3 = colp.reshape(NW, CPW, CH)

    deg_parts = _deg_kernel(row2, w2)
    dinv = _dinv_tc(deg_parts.reshape(NW, NPAD))
    norm = _norm_kernel(dinv, row2, col2, w2)
    normi = lax.bitcast_convert_type(norm, jnp.int32).reshape(NW, CPW, 1, CH)
    packed = jnp.concatenate(
        [row3.reshape(NW, CPW, 1, CH), col3.reshape(NW, CPW, 1, CH), normi],
        axis=2)
    e1, e2, e3 = _elin_tc(eap, We1.T, r2(be1), We2.T, r2(be2), We3.T, r2(be3))

    h1 = _nlin_tc(x, W1.T, r2(b1))
    p1 = _layer_kernel(h1, e1, packed, zero_nd)
    st1, _ = _nstats_tc(p1)
    h2 = _comb2_tc(p1, st1, r2(g1), r2(bt1), W2.T, r2(b2))
    p2 = _layer_kernel(h2, e2, packed, zero_nd)
    st2, _ = _nstats_tc(p2)
    h3 = _comb2_tc(p2, st2, r2(g2), r2(bt2), W3.T, r2(b3))
    p3 = _layer_kernel(h3, e3, packed, zero_nd)
    st3, nx3 = _nstats_tc(p3)
    node = _comb2_tc(p3, st3, r2(g3), r2(bt3), Wn.T, r2(bnb))

    s_mat = _pair_kernel(nx3, row3, col3)
    new_e, est = _newe_tc(s_mat, e3, Wu.T, r2(bu))
    edge = _comb1_tc(new_e, est, r2(ge), r2(bte), Wle.T, r2(ble), E)
    return (node, edge)


# speedup vs baseline: 3.0596x; 1.1286x over previous
import functools

import jax
import jax.numpy as jnp
from jax import lax
from jax.experimental import pallas as pl
from jax.experimental.pallas import tpu as pltpu
from jax.experimental.pallas import tpu_sc as plsc

N = 10000
E = 320000
D = 128
DE = 16

NC = 2
NS = 16
NW = NC * NS
CH = 48
CPW = 212
EW = CPW * CH
EPAD = NW * EW
NPAD = 10240
ZST = 1000
EPS = 1e-5

_MESH = plsc.VectorSubcoreMesh(core_axis_name="c", subcore_axis_name="s")
_SC_PARAMS = pltpu.CompilerParams(needs_layout_passes=False)


@functools.partial(
    pl.kernel,
    out_type=jax.ShapeDtypeStruct((NW * NPAD,), jnp.float32),
    mesh=_MESH,
    compiler_params=_SC_PARAMS,
    scratch_types=[
        pltpu.VMEM((NPAD,), jnp.float32),
        pltpu.VMEM((EW,), jnp.int32),
        pltpu.VMEM((EW,), jnp.float32),
    ],
)
def _deg_kernel(row_hbm, w_hbm, out_hbm, deg_v, idx_v, w_v):
    cid = lax.axis_index("c")
    sid = lax.axis_index("s")
    wid = sid * NC + cid

    def zero_body(i, _):
        deg_v[pl.ds(i * 16, 16)] = jnp.zeros((16,), jnp.float32)
        return 0
    lax.fori_loop(0, NPAD // 16, zero_body, 0)

    pltpu.sync_copy(row_hbm.at[wid], idx_v)
    pltpu.sync_copy(w_hbm.at[wid], w_v)

    def grp(g, _):
        ids = idx_v[pl.ds(g * 16, 16)]
        vals = w_v[pl.ds(g * 16, 16)]
        plsc.addupdate_scatter(deg_v, [ids], vals)
        return 0
    lax.fori_loop(0, EW // 16, grp, 0)
    pltpu.sync_copy(deg_v, out_hbm.at[pl.ds(wid * NPAD, NPAD)])


@functools.partial(
    pl.kernel,
    out_type=jax.ShapeDtypeStruct((NW, EW), jnp.float32),
    mesh=_MESH,
    compiler_params=_SC_PARAMS,
    scratch_types=[
        pltpu.VMEM((NPAD,), jnp.float32),
        pltpu.VMEM((EW,), jnp.int32),
        pltpu.VMEM((EW,), jnp.int32),
        pltpu.VMEM((EW,), jnp.float32),
        pltpu.VMEM((EW,), jnp.float32),
    ],
)
def _norm_kernel(dinv_hbm, row_hbm, col_hbm, w_hbm, out_hbm,
                 dinv_v, r_v, c_v, w_v, o_v):
    cid = lax.axis_index("c")
    sid = lax.axis_index("s")
    wid = sid * NC + cid
    pltpu.sync_copy(dinv_hbm, dinv_v)
    pltpu.sync_copy(row_hbm.at[wid], r_v)
    pltpu.sync_copy(col_hbm.at[wid], c_v)
    pltpu.sync_copy(w_hbm.at[wid], w_v)

    def grp(g, _):
        a = plsc.load_gather(dinv_v, [r_v[pl.ds(g * 16, 16)]])
        b = plsc.load_gather(dinv_v, [c_v[pl.ds(g * 16, 16)]])
        o_v[pl.ds(g * 16, 16)] = a * b * w_v[pl.ds(g * 16, 16)]
        return 0
    lax.fori_loop(0, EW // 16, grp, 0)
    pltpu.sync_copy(o_v, out_hbm.at[wid])


@functools.partial(
    pl.kernel,
    out_type=jax.ShapeDtypeStruct((NC, N, D), jnp.float32),
    mesh=_MESH,
    compiler_params=_SC_PARAMS,
    scratch_types=[
        pltpu.VMEM_SHARED((N, D), jnp.float32),
        pltpu.VMEM((3, CH), jnp.int32),
        pltpu.VMEM((3, CH), jnp.int32),
        pltpu.VMEM((3, CH), jnp.int32),
        pltpu.VMEM((3, CH), jnp.int32),
        pltpu.VMEM((CH, D), jnp.float32),
        pltpu.VMEM((CH, D), jnp.float32),
        pltpu.VMEM((CH, D), jnp.float32),
        pltpu.VMEM((CH, D), jnp.float32),
        pltpu.VMEM((CH, D), jnp.float32),
        pltpu.VMEM((CH, D), jnp.float32),
        pltpu.SemaphoreType.DMA,
        pltpu.SemaphoreType.DMA,
        pltpu.SemaphoreType.DMA,
        pltpu.SemaphoreType.DMA,
        pltpu.SemaphoreType.DMA,
        pltpu.SemaphoreType.DMA,
        pltpu.SemaphoreType.DMA,
        pltpu.SemaphoreType.DMA,
        pltpu.SemaphoreType.DMA,
        pltpu.SemaphoreType.DMA,
    ],
)
def _layer_kernel(h_hbm, e_hbm, pk_hbm, zero_hbm, out_hbm,
                  acc_sh, pk0, pk1, pk2, pk3, e_a, e_b, g_a, g_b, m_a, m_b,
                  sp0, sp1, sp2, sp3, sea, seb, sga, sgb, ssa, ssb):
    cid = lax.axis_index("c")
    sid = lax.axis_index("s")
    wid = sid * NC + cid
    wbase = wid * EW
    @pl.when(sid < 10)
    def _():
        pltpu.sync_copy(zero_hbm.at[pl.ds(sid * ZST, ZST)],
                        acc_sh.at[pl.ds(sid * ZST, ZST)])
    plsc.subcore_barrier()

    pks = (pk0, pk1, pk2, pk3)
    sps = (sp0, sp1, sp2, sp3)
    slots = ((e_a, g_a, m_a, sea, sga, ssa),
             (e_b, g_b, m_b, seb, sgb, ssb))

    def pk_fetch(ck, qi):
        pltpu.async_copy(pk_hbm.at[wid, ck], pks[qi], sps[qi])

    def pk_wait(ck, qi):
        pltpu.make_async_copy(pk_hbm.at[wid, ck], pks[qi], sps[qi]).wait()

    def eg_fetch(ck, qi, j):
        ebuf, gbuf, _, se, sg, _ = slots[j]
        pltpu.async_copy(e_hbm.at[pl.ds(wbase + ck * CH, CH)], ebuf, se)
        pltpu.async_copy(h_hbm.at[pks[qi].at[0]], gbuf, sg)

    pk_fetch(0, 0)
    pk_fetch(1, 1)
    pk_wait(0, 0)
    eg_fetch(0, 0, 0)

    def quad(k, _):
        for q in range(4):
            ck = 4 * k + q
            j = q % 2
            ebuf, gbuf, mbuf, se, sg, ss = slots[j]
            @pl.when(ck >= 2)
            def _():
                pltpu.make_async_copy(
                    mbuf, acc_sh.at[pks[(q + 2) % 4].at[1]], ss).wait()
            @pl.when(ck + 1 < CPW)
            def _():
                pk_wait(ck + 1, (q + 1) % 4)
                eg_fetch(ck + 1, (q + 1) % 4, (j + 1) % 2)
            @pl.when(ck + 2 < CPW)
            def _():
                pk_fetch(ck + 2, (q + 2) % 4)
            pltpu.make_async_copy(
                e_hbm.at[pl.ds(wbase + ck * CH, CH)], ebuf, se).wait()
            pltpu.make_async_copy(h_hbm.at[pks[q].at[0]], gbuf, sg).wait()

            def grp(g, _):
                nvi = pks[q][2, pl.ds(g * 16, 16)]
                nvec = plsc.bitcast(nvi, jnp.float32)
                for i16 in range(16):
                    i = g * 16 + i16
                    s = nvec[i16]
                    for v in range(D // 16):
                        hv = gbuf[i, pl.ds(v * 16, 16)]
                        ev = ebuf[i, pl.ds(v * 16, 16)]
                        mbuf[i, pl.ds(v * 16, 16)] = (
                            jnp.maximum(hv + ev, 0.0) * s)
                return 0
            lax.fori_loop(0, CH // 16, grp, 0)
            pltpu.make_async_copy(
                mbuf, acc_sh.at[pks[q].at[1]], ss).start(add=True)
        return 0
    lax.fori_loop(0, CPW // 4, quad, 0)
    pltpu.make_async_copy(m_a, acc_sh.at[pks[2].at[1]], ssa).wait()
    pltpu.make_async_copy(m_b, acc_sh.at[pks[3].at[1]], ssb).wait()
    plsc.subcore_barrier()

    @pl.when(sid == 0)
    def _():
        pltpu.sync_copy(acc_sh, out_hbm.at[cid])


@functools.partial(
    pl.kernel,
    out_type=jax.ShapeDtypeStruct((EPAD, D), jnp.float32),
    mesh=_MESH,
    compiler_params=_SC_PARAMS,
    scratch_types=[
        pltpu.VMEM((CPW, CH), jnp.int32),
        pltpu.VMEM((CPW, CH), jnp.int32),
        pltpu.VMEM((CH, D), jnp.float32),
        pltpu.VMEM((CH, D), jnp.float32),
        pltpu.VMEM((CH, D), jnp.float32),
        pltpu.VMEM((CH, D), jnp.float32),
        pltpu.VMEM((CH, D), jnp.float32),
        pltpu.VMEM((CH, D), jnp.float32),
        pltpu.SemaphoreType.DMA,
        pltpu.SemaphoreType.DMA,
        pltpu.SemaphoreType.DMA,
        pltpu.SemaphoreType.DMA,
        pltpu.SemaphoreType.DMA,
        pltpu.SemaphoreType.DMA,
    ],
)
def _pair_kernel(nx_hbm, row_hbm, col_hbm, out_hbm,
                 ri_v, ci_v, a_a, a_b, b_a, b_b, o_a, o_b,
                 sra, srb, sca, scb, soa, sob):
    cid = lax.axis_index("c")
    sid = lax.axis_index("s")
    wid = sid * NC + cid
    wbase = wid * EW
    pltpu.sync_copy(row_hbm.at[wid], ri_v)
    pltpu.sync_copy(col_hbm.at[wid], ci_v)

    slots = ((a_a, b_a, o_a, sra, sca, soa),
             (a_b, b_b, o_b, srb, scb, sob))

    def fetch(ck, abuf, bbuf, sr, sc):
        pltpu.async_copy(nx_hbm.at[ri_v.at[ck]], abuf, sr)
        pltpu.async_copy(nx_hbm.at[ci_v.at[ck]], bbuf, sc)

    fetch(0, a_a, b_a, sra, sca)
    fetch(1, a_b, b_b, srb, scb)

    def pair(k, _):
        for j in (0, 1):
            abuf, bbuf, obuf, sr, sc, so = slots[j]
            ck = 2 * k + j
            pltpu.make_async_copy(nx_hbm.at[ri_v.at[ck]], abuf, sr).wait()
            pltpu.make_async_copy(nx_hbm.at[ci_v.at[ck]], bbuf, sc).wait()
            @pl.when(ck >= 2)
            def _():
                pltpu.make_async_copy(
                    obuf, out_hbm.at[pl.ds(wbase + (ck - 2) * CH, CH)],
                    so).wait()

            def edge(i, _):
                for v in range(D // 16):
                    obuf[i, pl.ds(v * 16, 16)] = (
                        abuf[i, pl.ds(v * 16, 16)]
                        + bbuf[i, pl.ds(v * 16, 16)])
                return 0
            lax.fori_loop(0, CH, edge, 0)
            pltpu.async_copy(
                obuf, out_hbm.at[pl.ds(wbase + ck * CH, CH)], so)
            @pl.when(ck + 2 < CPW)
            def _():
                fetch(ck + 2, abuf, bbuf, sr, sc)
        return 0
    lax.fori_loop(0, CPW // 2, pair, 0)
    pltpu.make_async_copy(
        o_a, out_hbm.at[pl.ds(wbase + (CPW - 2) * CH, CH)], soa).wait()
    pltpu.make_async_copy(
        o_b, out_hbm.at[pl.ds(wbase + (CPW - 1) * CH, CH)], sob).wait()


def _dinv_tc(deg_parts):
    def body(d_ref, o_ref):
        o_ref[...] = lax.rsqrt(jnp.sum(d_ref[...], axis=0) + 1.0)
    return pl.pallas_call(
        body,
        out_shape=jax.ShapeDtypeStruct((NPAD,), jnp.float32),
    )(deg_parts)


def _elin_tc(ea, w1t, b1, w2t, b2, w3t, b3):
    B = 512
    def body(ea_ref, w1_ref, b1_ref, w2_ref, b2_ref, w3_ref, b3_ref,
             o1_ref, o2_ref, o3_ref):
        ea_b = ea_ref[...]
        o1_ref[...] = jnp.dot(ea_b, w1_ref[...],
                              preferred_element_type=jnp.float32) + b1_ref[...]
        o2_ref[...] = jnp.dot(ea_b, w2_ref[...],
                              preferred_element_type=jnp.float32) + b2_ref[...]
        o3_ref[...] = jnp.dot(ea_b, w3_ref[...],
                              preferred_element_type=jnp.float32) + b3_ref[...]
    full = lambda s: pl.BlockSpec(s, lambda i: (0, 0))
    return pl.pallas_call(
        body,
        grid=(EPAD // B,),
        in_specs=[pl.BlockSpec((B, DE), lambda i: (i, 0)),
                  full((DE, D)), full((1, D)),
                  full((DE, D)), full((1, D)),
                  full((DE, D)), full((1, D))],
        out_specs=[pl.BlockSpec((B, D), lambda i: (i, 0))] * 3,
        out_shape=[jax.ShapeDtypeStruct((EPAD, D), jnp.float32)] * 3,
    )(ea, w1t, b1, w2t, b2, w3t, b3)


def _nlin_tc(x, wt, b):
    B = 1000
    def body(x_ref, w_ref, b_ref, o_ref):
        o_ref[...] = jnp.dot(x_ref[...], w_ref[...],
                             preferred_element_type=jnp.float32) + b_ref[...]
    return pl.pallas_call(
        body,
        grid=(N // B,),
        in_specs=[pl.BlockSpec((B, x.shape[1]), lambda i: (i, 0)),
                  pl.BlockSpec((x.shape[1], D), lambda i: (0, 0)),
                  pl.BlockSpec((1, D), lambda i: (0, 0))],
        out_specs=pl.BlockSpec((B, D), lambda i: (i, 0)),
        out_shape=jax.ShapeDtypeStruct((N, D), jnp.float32),
    )(x, wt, b)


def _nstats_tc(p):
    B = 1000
    def body(p_ref, st_ref, nx_ref):
        i = pl.program_id(0)
        z = p_ref[0] + p_ref[1]
        nx_ref[...] = z
        r = jnp.maximum(z, 0.0)
        st = jnp.concatenate([jnp.sum(r, axis=0, keepdims=True),
                              jnp.sum(r * r, axis=0, keepdims=True)], axis=0)
        @pl.when(i == 0)
        def _():
            st_ref[...] = st
        @pl.when(i > 0)
        def _():
            st_ref[...] = st_ref[...] + st
    return pl.pallas_call(
        body,
        grid=(N // B,),
        in_specs=[pl.BlockSpec((NC, B, D), lambda i: (0, i, 0))],
        out_specs=[pl.BlockSpec((2, D), lambda i: (0, 0)),
                   pl.BlockSpec((B, D), lambda i: (i, 0))],
        out_shape=[jax.ShapeDtypeStruct((2, D), jnp.float32),
                   jax.ShapeDtypeStruct((N, D), jnp.float32)],
    )(p)


def _comb2_tc(p, st, g, bt, wt, b):
    B = 1000
    def body(p_ref, st_ref, g_ref, bt_ref, w_ref, b_ref, o_ref):
        r = jnp.maximum(p_ref[0] + p_ref[1], 0.0)
        m = st_ref[0:1] * (1.0 / N)
        var = st_ref[1:2] * (1.0 / N) - m * m
        a = g_ref[...] * lax.rsqrt(var + EPS)
        c = bt_ref[...] - m * a
        o_ref[...] = jnp.dot(r * a + c, w_ref[...],
                             preferred_element_type=jnp.float32) + b_ref[...]
    full = lambda s: pl.BlockSpec(s, lambda i: (0, 0))
    return pl.pallas_call(
        body,
        grid=(N // B,),
        in_specs=[pl.BlockSpec((NC, B, D), lambda i: (0, i, 0)),
                  full((2, D)), full((1, D)), full((1, D)),
                  full((D, D)), full((1, D))],
        out_specs=pl.BlockSpec((B, D), lambda i: (i, 0)),
        out_shape=jax.ShapeDtypeStruct((N, D), jnp.float32),
    )(p, st, g, bt, wt, b)


def _newe_tc(s_mat, e3, wut, bu):
    B = 512
    def body(s_ref, e_ref, w_ref, b_ref, o_ref, st_ref):
        i = pl.program_id(0)
        z = jnp.dot(s_ref[...], w_ref[...],
                    preferred_element_type=jnp.float32) + b_ref[...] + e_ref[...]
        o_ref[...] = z
        r = jnp.maximum(z, 0.0)
        st = jnp.concatenate([jnp.sum(r, axis=0, keepdims=True),
                              jnp.sum(r * r, axis=0, keepdims=True)], axis=0)
        @pl.when(i == 0)
        def _():
            st_ref[...] = st
        @pl.when(i > 0)
        def _():
            st_ref[...] = st_ref[...] + st
    full = lambda s: pl.BlockSpec(s, lambda i: (0, 0))
    return pl.pallas_call(
        body,
        grid=(E // B,),
        in_specs=[pl.BlockSpec((B, D), lambda i: (i, 0)),
                  pl.BlockSpec((B, D), lambda i: (i, 0)),
                  full((D, D)), full((1, D))],
        out_specs=[pl.BlockSpec((B, D), lambda i: (i, 0)),
                   pl.BlockSpec((2, D), lambda i: (0, 0))],
        out_shape=[jax.ShapeDtypeStruct((E, D), jnp.float32),
                   jax.ShapeDtypeStruct((2, D), jnp.float32)],
    )(s_mat, e3, wut, bu)


def _comb1_tc(x, st, g, bt, wt, b, cnt):
    B = 512
    M = x.shape[0]
    def body(x_ref, st_ref, g_ref, bt_ref, w_ref, b_ref, o_ref):
        r = jnp.maximum(x_ref[...], 0.0)
        m = st_ref[0:1] * (1.0 / cnt)
        var = st_ref[1:2] * (1.0 / cnt) - m * m
        a = g_ref[...] * lax.rsqrt(var + EPS)
        c = bt_ref[...] - m * a
        o_ref[...] = jnp.dot(r * a + c, w_ref[...],
                             preferred_element_type=jnp.float32) + b_ref[...]
    full = lambda s: pl.BlockSpec(s, lambda i: (0, 0))
    return pl.pallas_call(
        body,
        grid=(M // B,),
        in_specs=[pl.BlockSpec((B, D), lambda i: (i, 0)),
                  full((2, D)), full((1, D)), full((1, D)),
                  full((D, D)), full((1, D))],
        out_specs=pl.BlockSpec((B, D), lambda i: (i, 0)),
        out_shape=jax.ShapeDtypeStruct((M, D), jnp.float32),
    )(x, st, g, bt, wt, b)


def kernel(x, edge_index, edge_attr, W1, b1, We1, be1, W2, b2, We2, be2,
           W3, b3, We3, be3, Wu, bu, g1, bt1, g2, bt2, g3, bt3, ge, bte,
           Wn, bnb, Wle, ble):
    pad = EPAD - E
    rowp = jnp.concatenate([edge_index[0], jnp.zeros((pad,), jnp.int32)])
    colp = jnp.concatenate([edge_index[1], jnp.zeros((pad,), jnp.int32)])
    w = jnp.concatenate([jnp.ones((E,), jnp.float32),
                         jnp.zeros((pad,), jnp.float32)])
    eap = jnp.concatenate([edge_attr, jnp.zeros((pad, DE), jnp.float32)], axis=0)
    zero_nd = jnp.zeros((N, D), jnp.float32)
    r2 = lambda v: v.reshape(1, D)
    row2 = rowp.reshape(NW, EW)
    col2 = colp.reshape(NW, EW)
    w2 = w.reshape(NW, EW)
    row3 = rowp.reshape(NW, CPW, CH)
    col3 = colp.reshape(NW, CPW, CH)

    deg_parts = _deg_kernel(row2, w2)
    dinv = _dinv_tc(deg_parts.reshape(NW, NPAD))
    norm = _norm_kernel(dinv, row2, col2, w2)
    normi = lax.bitcast_convert_type(norm, jnp.int32).reshape(NW, CPW, 1, CH)
    packed = jnp.concatenate(
        [row3.reshape(NW, CPW, 1, CH), col3.reshape(NW, CPW, 1, CH), normi],
        axis=2)
    e1, e2, e3 = _elin_tc(eap, We1.T, r2(be1), We2.T, r2(be2), We3.T, r2(be3))

    h1 = _nlin_tc(x, W1.T, r2(b1))
    p1 = _layer_kernel(h1, e1, packed, zero_nd)
    st1, _ = _nstats_tc(p1)
    h2 = _comb2_tc(p1, st1, r2(g1), r2(bt1), W2.T, r2(b2))
    p2 = _layer_kernel(h2, e2, packed, zero_nd)
    st2, _ = _nstats_tc(p2)
    h3 = _comb2_tc(p2, st2, r2(g2), r2(bt2), W3.T, r2(b3))
    p3 = _layer_kernel(h3, e3, packed, zero_nd)
    st3, nx3 = _nstats_tc(p3)
    node = _comb2_tc(p3, st3, r2(g3), r2(bt3), Wn.T, r2(bnb))

    s_mat = _pair_kernel(nx3, row3, col3)
    new_e, est = _newe_tc(s_mat, e3, Wu.T, r2(bu))
    edge = _comb1_tc(new_e, est, r2(ge), r2(bte), Wle.T, r2(ble), E)
    return (node, edge)
